# baseline (speedup 1.0000x reference)
import functools

import jax
import jax.numpy as jnp
from jax import lax
from jax.experimental import pallas as pl
from jax.experimental.pallas import tpu as pltpu
from jax.experimental.pallas import tpu_sc as plsc

_NC = 2
_NS = 16
_NW = _NC * _NS


def _sc_position_add(x, input_pos, emb_weight):
    B, S, D = x.shape
    BP = B // 2
    NSBLK = _NW // BP
    SPW = S // NSBLK
    CH = 16
    NCHUNK = SPW // CH
    LANES = 16

    mesh = plsc.VectorSubcoreMesh(core_axis_name="c", subcore_axis_name="s")

    @functools.partial(
        pl.kernel,
        mesh=mesh,
        out_type=jax.ShapeDtypeStruct((B, S, D), jnp.float32),
        scratch_types=[
            pltpu.VMEM((CH,), jnp.int32),
            pltpu.VMEM((2, CH, D), jnp.float32),
            pltpu.VMEM((CH, D), jnp.float32),
            pltpu.SemaphoreType.DMA,
            pltpu.SemaphoreType.DMA,
        ],
    )
    def body(x_hbm, pos_hbm, emb_hbm, out_hbm, idx_v, xbuf, ebuf, semx, seme):
        wid = lax.axis_index("s") * _NC + lax.axis_index("c")
        bp = wid // NSBLK
        sblk = wid % NSBLK
        s_base = sblk * SPW
        b0 = 2 * bp

        def chunk(i, carry):
            s0 = s_base + i * CH
            pltpu.sync_copy(pos_hbm.at[pl.ds(s0, CH)], idx_v)
            cx = pltpu.async_copy(
                x_hbm.at[pl.ds(b0, 2), pl.ds(s0, CH), :], xbuf, semx)
            ce = pltpu.async_copy(emb_hbm.at[idx_v], ebuf, seme)
            ce.wait()
            cx.wait()
            for r in range(CH):
                def kbody(k, c, _r=r):
                    off = k * LANES
                    e = ebuf[_r, pl.ds(off, LANES)]
                    plsc.addupdate(xbuf.at[0, _r, pl.ds(off, LANES)], e)
                    plsc.addupdate(xbuf.at[1, _r, pl.ds(off, LANES)], e)
                    return c
                lax.fori_loop(0, D // LANES, kbody, 0, unroll=8)
            pltpu.sync_copy(
                xbuf, out_hbm.at[pl.ds(b0, 2), pl.ds(s0, CH), :])
            return carry

        lax.fori_loop(0, NCHUNK, chunk, 0)

    return body(x, input_pos, emb_weight)


def kernel(x, input_pos, emb_weight):
    return _sc_position_add(x, input_pos, emb_weight)
